# baseline (speedup 1.0000x reference)
import functools

import jax
import jax.numpy as jnp
from jax import lax
from jax.experimental import pallas as pl
from jax.experimental.pallas import tpu as pltpu
from jax.experimental.pallas import tpu_sc as plsc

D_MODEL = 1024
SCALE = 32.0
NC, NS, L = 2, 16, 16
NW = NC * NS


@functools.lru_cache(maxsize=None)
def _make_emb(B: int):
    assert B % NW == 0
    b_per_w = B // NW
    chunk = 8
    nbuf = 8
    look = 6
    half = chunk // 2
    assert b_per_w % chunk == 0
    n_chunks = b_per_w // chunk
    assert n_chunks % nbuf == 0 and n_chunks > nbuf

    mesh = plsc.VectorSubcoreMesh(
        core_axis_name="c", subcore_axis_name="s",
        num_cores=NC, num_subcores=NS)

    @functools.partial(
        pl.kernel,
        out_type=jax.ShapeDtypeStruct((B, D_MODEL), jnp.float32),
        mesh=mesh,
        scratch_types=[
            pltpu.VMEM((b_per_w,), jnp.int32),
            pltpu.VMEM((nbuf, chunk, D_MODEL), jnp.float32),
            [pltpu.SemaphoreType.DMA] * nbuf,
            [pltpu.SemaphoreType.DMA] * nbuf,
        ],
    )
    def _emb(idx_hbm, table_hbm, out_hbm, idx_v, buf, gsems, ssems):
        wid = lax.axis_index("s") * NC + lax.axis_index("c")
        base = wid * b_per_w
        pltpu.sync_copy(idx_hbm.at[pl.ds(base, b_per_w)], idx_v)

        def gather(g, b):
            return pltpu.make_async_copy(
                table_hbm.at[idx_v.at[pl.ds(g * chunk, chunk)]],
                buf.at[b], gsems[b])

        def scatter_half(g, b, h):
            return pltpu.make_async_copy(
                buf.at[b].at[pl.ds(h * half, half)],
                out_hbm.at[pl.ds(base + g * chunk + h * half, half)],
                ssems[b])

        def scatter_wait(b):
            return pltpu.make_async_copy(
                buf.at[b], out_hbm.at[pl.ds(base, chunk)], ssems[b])

        for j in range(look):
            gather(j, j).start()

        @pl.loop(0, n_chunks, step=nbuf)
        def _outer(g0):
            for b in range(nbuf):
                g = g0 + b
                gather(g, b).wait()

                nb = (b + look) % nbuf

                @pl.when(jnp.logical_and(g + look < n_chunks,
                                         g >= nbuf - look))
                def _():
                    scatter_wait(nb).wait()

                @pl.when(g + look < n_chunks)
                def _():
                    gather(g + look, nb).start()

                for h in range(chunk // half):
                    @pl.loop(h * half, (h + 1) * half)
                    def _row(r):
                        for i in range(D_MODEL // L):
                            buf[b, r, pl.ds(i * L, L)] = (
                                buf[b, r, pl.ds(i * L, L)] * SCALE)

                    scatter_half(g, b, h).start()

        for b in range(nbuf):
            scatter_wait(b).wait()

    return _emb


def kernel(x, table):
    idx = x.reshape(-1).astype(jnp.int32)
    out = _make_emb(idx.shape[0])(idx, table)
    return out.reshape(x.shape + (D_MODEL,))


# speedup vs baseline: 1.0678x; 1.0678x over previous
import functools

import jax
import jax.numpy as jnp
from jax import lax
from jax.experimental import pallas as pl
from jax.experimental.pallas import tpu as pltpu
from jax.experimental.pallas import tpu_sc as plsc

D_MODEL = 1024
SCALE = 32.0
NC, NS, L = 2, 16, 16
NW = NC * NS


@functools.lru_cache(maxsize=None)
def _make_emb(B: int):
    assert B % NW == 0
    b_per_w = B // NW
    chunk = 8
    nbuf = 8
    look = 6
    half = chunk // 2
    assert b_per_w % chunk == 0
    n_chunks = b_per_w // chunk
    assert n_chunks % nbuf == 0 and n_chunks > nbuf

    mesh = plsc.VectorSubcoreMesh(
        core_axis_name="c", subcore_axis_name="s",
        num_cores=NC, num_subcores=NS)

    @functools.partial(
        pl.kernel,
        out_type=jax.ShapeDtypeStruct((B, D_MODEL), jnp.float32),
        mesh=mesh,
        scratch_types=[
            pltpu.VMEM((b_per_w,), jnp.int32),
            pltpu.VMEM((nbuf, chunk, D_MODEL), jnp.float32),
            [pltpu.SemaphoreType.DMA] * nbuf,
            [pltpu.SemaphoreType.DMA] * nbuf,
        ],
    )
    def _emb(idx_hbm, table_hbm, out_hbm, idx_v, buf, gsems, ssems):
        wid = lax.axis_index("s") * NC + lax.axis_index("c")
        base = wid * b_per_w
        pltpu.sync_copy(idx_hbm.at[pl.ds(base, b_per_w)], idx_v)

        def gather(g, b):
            return pltpu.make_async_copy(
                table_hbm.at[idx_v.at[pl.ds(g * chunk, chunk)]],
                buf.at[b], gsems[b])

        def scatter_full(g, b):
            return pltpu.make_async_copy(
                buf.at[b], out_hbm.at[pl.ds(base + g * chunk, chunk)],
                ssems[b])

        def scatter_wait(b):
            return pltpu.make_async_copy(
                buf.at[b], out_hbm.at[pl.ds(base, chunk)], ssems[b])

        for j in range(look):
            gather(j, j).start()

        @pl.loop(0, n_chunks, step=nbuf)
        def _outer(g0):
            for b in range(nbuf):
                g = g0 + b
                gather(g, b).wait()

                nb = (b + look) % nbuf

                @pl.when(jnp.logical_and(g + look < n_chunks,
                                         g >= nbuf - look))
                def _():
                    scatter_wait(nb).wait()

                @pl.when(g + look < n_chunks)
                def _():
                    gather(g + look, nb).start()

                @pl.loop(0, chunk)
                def _row(r):
                    for i in range(D_MODEL // L):
                        buf[b, r, pl.ds(i * L, L)] = (
                            buf[b, r, pl.ds(i * L, L)] * SCALE)

                scatter_full(g, b).start()

        for b in range(nbuf):
            scatter_wait(b).wait()

    return _emb


def kernel(x, table):
    idx = x.reshape(-1).astype(jnp.int32)
    out = _make_emb(idx.shape[0])(idx, table)
    return out.reshape(x.shape + (D_MODEL,))
